# baseline (speedup 1.0000x reference)
import functools

import jax
import jax.numpy as jnp
from jax import lax
from jax.experimental import pallas as pl
from jax.experimental.pallas import tpu as pltpu
from jax.experimental.pallas import tpu_sc as plsc

_B = 16384
_P = 64
_D = 128
_BLK = 64
_ROWS = 16


def _sc_lookup_body(rank_hbm, file_hbm, ranks_hbm, files_hbm, out_hbm,
                    idx_r, idx_f, rows_r, rows_f, sem_r, sem_f):
    wid = lax.axis_index("s") * 2 + lax.axis_index("c")

    @pl.when(wid < _P // _ROWS)
    def _():
        base = wid * _ROWS
        pltpu.sync_copy(ranks_hbm.at[pl.ds(base, _ROWS)], idx_r)
        pltpu.sync_copy(files_hbm.at[pl.ds(base, _ROWS)], idx_f)
        cp_r = pltpu.async_copy(rank_hbm.at[idx_r], rows_r, sem_r)
        cp_f = pltpu.async_copy(file_hbm.at[idx_f], rows_f, sem_f)
        cp_r.wait()
        cp_f.wait()
        for i in range(_ROWS):
            for j in range(_D // 16):
                sl = pl.ds(j * 16, 16)
                rows_f[i, sl] = rows_f[i, sl] + rows_r[i, sl]
        pltpu.sync_copy(rows_f, out_hbm.at[pl.ds(base, _ROWS)])


_sc_lookup = functools.partial(
    pl.kernel,
    out_type=jax.ShapeDtypeStruct((_P, _D), jnp.float32),
    mesh=plsc.VectorSubcoreMesh(core_axis_name="c", subcore_axis_name="s"),
    scratch_types=[
        pltpu.VMEM((_ROWS,), jnp.int32),
        pltpu.VMEM((_ROWS,), jnp.int32),
        pltpu.VMEM((_ROWS, _D), jnp.float32),
        pltpu.VMEM((_ROWS, _D), jnp.float32),
        pltpu.SemaphoreType.DMA,
        pltpu.SemaphoreType.DMA,
    ],
)(_sc_lookup_body)


def _tc_broadcast_body(table_ref, out_ref):
    table = table_ref[...]
    out_ref[...] = jnp.broadcast_to(table[None, :, :], (_BLK, _P, _D))


def kernel(rank_embed, file_embed, batch_size):
    positions = jnp.arange(_P, dtype=jnp.int32)
    table = _sc_lookup(rank_embed, file_embed, positions // 8, positions % 8)
    return pl.pallas_call(
        _tc_broadcast_body,
        grid=(_B // _BLK,),
        in_specs=[pl.BlockSpec((_P, _D), lambda i: (0, 0))],
        out_specs=pl.BlockSpec((_BLK, _P, _D), lambda i: (i, 0, 0)),
        out_shape=jax.ShapeDtypeStruct((_B, _P, _D), jnp.float32),
    )(table)


# speedup vs baseline: 1.1762x; 1.1762x over previous
import functools

import jax
import jax.numpy as jnp
from jax import lax
from jax.experimental import pallas as pl
from jax.experimental.pallas import tpu as pltpu
from jax.experimental.pallas import tpu_sc as plsc

_B = 16384
_P = 64
_D = 128
_BLK = 128
_B1 = 2048
_ROWS = 16


def _sc_lookup_body(rank_hbm, file_hbm, ranks_hbm, files_hbm, out_hbm,
                    idx_r, idx_f, rows_r, rows_f, sem_r, sem_f):
    wid = lax.axis_index("s") * 2 + lax.axis_index("c")

    @pl.when(wid < _P // _ROWS)
    def _():
        base = wid * _ROWS
        pltpu.sync_copy(ranks_hbm.at[pl.ds(base, _ROWS)], idx_r)
        pltpu.sync_copy(files_hbm.at[pl.ds(base, _ROWS)], idx_f)
        cp_r = pltpu.async_copy(rank_hbm.at[idx_r], rows_r, sem_r)
        cp_f = pltpu.async_copy(file_hbm.at[idx_f], rows_f, sem_f)
        cp_r.wait()
        cp_f.wait()
        for i in range(_ROWS):
            for j in range(_D // 16):
                sl = pl.ds(j * 16, 16)
                rows_f[i, sl] = rows_f[i, sl] + rows_r[i, sl]
        pltpu.sync_copy(rows_f, out_hbm.at[pl.ds(base, _ROWS)])


_sc_lookup = functools.partial(
    pl.kernel,
    out_type=jax.ShapeDtypeStruct((_P, _D), jnp.float32),
    mesh=plsc.VectorSubcoreMesh(core_axis_name="c", subcore_axis_name="s"),
    scratch_types=[
        pltpu.VMEM((_ROWS,), jnp.int32),
        pltpu.VMEM((_ROWS,), jnp.int32),
        pltpu.VMEM((_ROWS, _D), jnp.float32),
        pltpu.VMEM((_ROWS, _D), jnp.float32),
        pltpu.SemaphoreType.DMA,
        pltpu.SemaphoreType.DMA,
    ],
)(_sc_lookup_body)


def _tc_head_body(rank_ref, file_ref, out_ref):
    r = rank_ref[...]
    f = file_ref[...]
    rank_part = jnp.broadcast_to(r[:, None, :], (8, 8, _D))
    file_part = jnp.broadcast_to(f[None, :, :], (8, 8, _D))
    table = (rank_part + file_part).reshape(_P, _D)
    out_ref[...] = jnp.broadcast_to(table[None, :, :], (_BLK, _P, _D))


def _tc_tail_body(partial_hbm, table_ref, out_ref):
    del partial_hbm
    out_ref[...] = jnp.broadcast_to(table_ref[...][None, :, :], (_BLK, _P, _D))


def kernel(rank_embed, file_embed, batch_size):
    positions = jnp.arange(_P, dtype=jnp.int32)
    table = _sc_lookup(rank_embed, file_embed, positions // 8, positions % 8)
    head = pl.pallas_call(
        _tc_head_body,
        grid=(_B1 // _BLK,),
        in_specs=[
            pl.BlockSpec((8, _D), lambda i: (0, 0)),
            pl.BlockSpec((8, _D), lambda i: (0, 0)),
        ],
        out_specs=pl.BlockSpec((_BLK, _P, _D), lambda i: (i, 0, 0)),
        out_shape=jax.ShapeDtypeStruct((_B, _P, _D), jnp.float32),
    )(rank_embed, file_embed)
    return pl.pallas_call(
        _tc_tail_body,
        grid=((_B - _B1) // _BLK,),
        in_specs=[
            pl.BlockSpec(memory_space=pltpu.MemorySpace.HBM),
            pl.BlockSpec((_P, _D), lambda i: (0, 0)),
        ],
        out_specs=pl.BlockSpec(
            (_BLK, _P, _D), lambda i: (i + _B1 // _BLK, 0, 0)),
        out_shape=jax.ShapeDtypeStruct((_B, _P, _D), jnp.float32),
        input_output_aliases={0: 0},
    )(head, table)
